# baseline (speedup 1.0000x reference)
import functools

import jax
import jax.numpy as jnp
from jax import lax
from jax.experimental import pallas as pl
from jax.experimental.pallas import tpu as pltpu
from jax.experimental.pallas import tpu_sc as plsc

_B, _L, _D = 128, 576, 1024
_K = _L // 2
_N = _B * _K
_NW = 32
_EPT = _N // _NW
_CHUNK = 128
_NCH = _EPT // _CHUNK


def _keep_flat_indices():
    noise = jax.random.normal(jax.random.key(42), (_B, _L), dtype=jnp.float32)
    ids = jnp.argsort(noise, axis=1)[:, :_K].astype(jnp.int32)
    flat = (jnp.arange(_B, dtype=jnp.int32)[:, None] * _L + ids) * _D
    return flat.reshape(_NW, _NCH, _CHUNK)


def _gather_body(table_hbm, idx_hbm, out_hbm, idx_v, out_v, sem):
    wid = lax.axis_index("s") * 2 + lax.axis_index("c")
    pltpu.sync_copy(idx_hbm.at[wid], idx_v)
    copies = [
        pltpu.async_copy(table_hbm.at[idx_v.at[j]],
                         out_v.at[pl.ds(j * _CHUNK, _CHUNK)], sem)
        for j in range(_NCH)
    ]
    for c in copies:
        c.wait()
    pltpu.sync_copy(out_v, out_hbm.at[pl.ds(wid * _EPT, _EPT)])


@functools.lru_cache(maxsize=None)
def _gather():
    return functools.partial(
        pl.kernel,
        mesh=plsc.VectorSubcoreMesh(core_axis_name="c", subcore_axis_name="s"),
        out_type=jax.ShapeDtypeStruct((_N,), jnp.float32),
        scratch_types=[
            pltpu.VMEM((_NCH, _CHUNK), jnp.int32),
            pltpu.VMEM((_EPT,), jnp.float32),
            pltpu.SemaphoreType.DMA,
        ],
    )(_gather_body)


def kernel(image_features):
    table = image_features.reshape(_B * _L * _D)
    idx = _keep_flat_indices()
    flat = _gather()(table, idx)
    return flat.reshape(_B, _K, 1)


# speedup vs baseline: 4.6154x; 4.6154x over previous
import functools

import jax
import jax.numpy as jnp
from jax import lax
from jax.experimental import pallas as pl
from jax.experimental.pallas import tpu as pltpu
from jax.experimental.pallas import tpu_sc as plsc

_B, _L, _D = 128, 576, 1024
_K = _L // 2
_N = _B * _K
_NW = 32
_EPT = _N // _NW
_CHUNK = 128
_NCH = _EPT // _CHUNK


def _keep_flat_indices():
    noise = jax.random.normal(jax.random.key(42), (_B, _L), dtype=jnp.float32)
    ids = jnp.argsort(noise, axis=1)[:, :_K].astype(jnp.int32)
    b = jnp.arange(_B, dtype=jnp.int32)[:, None]
    flat = b * (_L * _D) + (ids // 8) * (8 * _D) + (ids % 8) * 128
    return flat.reshape(_NW, _NCH, _CHUNK)


def _gather_body(table_hbm, idx_hbm, out_hbm, idx_v, out_v, sem):
    wid = lax.axis_index("s") * 2 + lax.axis_index("c")
    pltpu.sync_copy(idx_hbm.at[wid], idx_v)
    copies = [
        pltpu.async_copy(table_hbm.at[idx_v.at[j]],
                         out_v.at[pl.ds(j * _CHUNK, _CHUNK)], sem)
        for j in range(_NCH)
    ]
    for c in copies:
        c.wait()
    pltpu.sync_copy(out_v, out_hbm.at[pl.ds(wid * _EPT, _EPT)])


@functools.lru_cache(maxsize=None)
def _gather():
    return functools.partial(
        pl.kernel,
        mesh=plsc.VectorSubcoreMesh(core_axis_name="c", subcore_axis_name="s"),
        out_type=jax.ShapeDtypeStruct((_N,), jnp.float32),
        scratch_types=[
            pltpu.VMEM((_NCH, _CHUNK), jnp.int32),
            pltpu.VMEM((_EPT,), jnp.float32),
            pltpu.SemaphoreType.DMA,
        ],
    )(_gather_body)


def kernel(image_features):
    tiles = image_features.reshape(_B, _L // 8, 8, _D // 128, 128)
    table = tiles.transpose(0, 1, 3, 2, 4).reshape(_B * _L * _D)
    idx = _keep_flat_indices()
    flat = _gather()(table, idx)
    return flat.reshape(_B, _K, 1)


# speedup vs baseline: 9.9013x; 2.1453x over previous
import functools

import numpy as np
import jax
import jax.numpy as jnp
from jax import lax
from jax.experimental import pallas as pl
from jax.experimental.pallas import tpu as pltpu
from jax.experimental.pallas import tpu_sc as plsc

_B, _L, _D = 128, 576, 1024
_K = _L // 2
_N = _B * _K
_NW = 32
_EPT = _N // _NW
_CHUNK = 128
_NCH = _EPT // _CHUNK


def _keep_flat_indices():
    with jax.default_device(jax.local_devices(backend="cpu")[0]):
        noise = jax.random.normal(
            jax.random.key(42), (_B, _L), dtype=jnp.float32)
        ids = np.asarray(jnp.argsort(noise, axis=1)[:, :_K], dtype=np.int64)
    b = np.arange(_B, dtype=np.int64)[:, None]
    flat = b * (_L * _D) + (ids // 8) * (8 * _D) + (ids % 8) * 128
    return flat.reshape(_NW, _NCH, _CHUNK).astype(np.int32)


_IDX = _keep_flat_indices()


def _gather_body(table_hbm, idx_hbm, out_hbm, idx_v, out_v, sem):
    wid = lax.axis_index("s") * 2 + lax.axis_index("c")
    pltpu.sync_copy(idx_hbm.at[wid], idx_v)
    copies = [
        pltpu.async_copy(table_hbm.at[idx_v.at[j]],
                         out_v.at[pl.ds(j * _CHUNK, _CHUNK)], sem)
        for j in range(_NCH)
    ]
    for c in copies:
        c.wait()
    pltpu.sync_copy(out_v, out_hbm.at[pl.ds(wid * _EPT, _EPT)])


@functools.lru_cache(maxsize=None)
def _gather():
    return functools.partial(
        pl.kernel,
        mesh=plsc.VectorSubcoreMesh(core_axis_name="c", subcore_axis_name="s"),
        out_type=jax.ShapeDtypeStruct((_N,), jnp.float32),
        scratch_types=[
            pltpu.VMEM((_NCH, _CHUNK), jnp.int32),
            pltpu.VMEM((_EPT,), jnp.float32),
            pltpu.SemaphoreType.DMA,
        ],
    )(_gather_body)


def kernel(image_features):
    tiles = image_features.reshape(_B, _L // 8, 8, _D // 128, 128)
    table = tiles.transpose(0, 1, 3, 2, 4).reshape(_B * _L * _D)
    flat = _gather()(table, _IDX)
    return flat.reshape(_B, _K, 1)


# speedup vs baseline: 10.7961x; 1.0904x over previous
import functools

import numpy as np
import jax
import jax.numpy as jnp
from jax import lax
from jax.experimental import pallas as pl
from jax.experimental.pallas import tpu as pltpu
from jax.experimental.pallas import tpu_sc as plsc

_B, _L, _D = 128, 576, 1024
_K = _L // 2
_N = _B * _K
_NW = 32
_EPT = _N // _NW
_CHUNK = _B
_NCH = _EPT // _CHUNK


def _keep_flat_indices():
    with jax.default_device(jax.local_devices(backend="cpu")[0]):
        noise = jax.random.normal(
            jax.random.key(42), (_B, _L), dtype=jnp.float32)
        ids = np.asarray(jnp.argsort(noise, axis=1)[:, :_K], dtype=np.int64)
    b = np.arange(_B, dtype=np.int64)[:, None]
    offs = b * (_L * _D) + (ids // 8) * (8 * _D) + (ids % 8) * 128
    idx = np.zeros((_NW, 16, _CHUNK), dtype=np.int32)
    idx[:, :_NCH, :] = offs.T.reshape(_NW, _NCH, _CHUNK)
    return idx


_IDX = _keep_flat_indices()


def _gather_body(table_hbm, idx_hbm, out_hbm, idx_v, out_v, sem):
    wid = lax.axis_index("s") * 2 + lax.axis_index("c")
    pltpu.sync_copy(idx_hbm.at[wid], idx_v)
    copies = [
        pltpu.async_copy(table_hbm.at[idx_v.at[j]],
                         out_v.at[pl.ds(j * _CHUNK, _CHUNK)], sem)
        for j in range(_NCH)
    ]
    for c in copies:
        c.wait()
    pltpu.sync_copy(out_v, out_hbm.at[pl.ds(wid * _EPT, _EPT)])


@functools.lru_cache(maxsize=None)
def _gather():
    return functools.partial(
        pl.kernel,
        mesh=plsc.VectorSubcoreMesh(core_axis_name="c", subcore_axis_name="s"),
        out_type=jax.ShapeDtypeStruct((_N,), jnp.float32),
        scratch_types=[
            pltpu.VMEM((16, _CHUNK), jnp.int32),
            pltpu.VMEM((_EPT,), jnp.float32),
            pltpu.SemaphoreType.DMA,
        ],
    )(_gather_body)


def kernel(image_features):
    tiles = image_features.reshape(_B, _L // 8, 8, _D // 128, 128)
    table = tiles.transpose(0, 1, 3, 2, 4).reshape(_B * _L * _D)
    z = _gather()(table, _IDX)
    return z.reshape(_K, 1, _B).transpose(2, 0, 1)
